# baseline (speedup 1.0000x reference)
import functools

import jax
import jax.numpy as jnp
from jax import lax
from jax.experimental import pallas as pl
from jax.experimental.pallas import tpu as pltpu
from jax.experimental.pallas import tpu_sc as plsc

B = 16384
F = 100
D = 32
BPW = 512
TCW = BPW // 128
TBLK = TCW * 8 * 128
FPLANE = (D // 8) * (B // 128) * 8 * 128


def _build_call(num_cores, num_subcores):
    mesh = plsc.VectorSubcoreMesh(core_axis_name="c", subcore_axis_name="s")

    @functools.partial(
        pl.kernel,
        mesh=mesh,
        out_type=jax.ShapeDtypeStruct((F * FPLANE,), jnp.float32),
        scratch_types=[
            pltpu.VMEM((BPW, F), jnp.int32),
            pltpu.VMEM((BPW,), jnp.int32),
            pltpu.VMEM((BPW,), jnp.int32),
            pltpu.VMEM((BPW, D), jnp.float32),
            pltpu.VMEM((BPW, D), jnp.float32),
            pltpu.VMEM(((D // 8) * TBLK,), jnp.float32),
            pltpu.VMEM(((D // 8) * TBLK,), jnp.float32),
            pltpu.SemaphoreType.DMA,
            pltpu.SemaphoreType.DMA,
            pltpu.SemaphoreType.DMA,
            pltpu.SemaphoreType.DMA,
        ],
        compiler_params=pltpu.CompilerParams(
            use_tc_tiling_on_sc=False, needs_layout_passes=False
        ),
    )
    def k(x_hbm, tab_hbm, out_hbm, xblk, if0, if1, g0, g1, t0, t1, gs0, gs1, ws0, ws1):
        wid = lax.axis_index("s") * num_cores + lax.axis_index("c")
        b0 = pl.multiple_of(wid * BPW, BPW)
        tc0 = wid * TCW
        IF = (if0, if1)
        G = (g0, g1)
        T = (t0, t1)
        gsem = (gs0, gs1)
        wsem = (ws0, ws1)

        iota16 = lax.iota(jnp.int32, 16)

        def extract(f, p):
            dst = IF[p]
            fvec = iota16 * 0 + f

            for j in range(BPW // 16):
                rows = j * 16 + iota16
                v = plsc.load_gather(xblk, [rows, fvec])
                plsc.store_scatter(dst, [rows], v)

        def start_gather(p):
            pltpu.async_copy(tab_hbm.at[IF[p]], G[p], gsem[p])

        def wait_gather(p):
            pltpu.make_async_copy(tab_hbm.at[IF[p]], G[p], gsem[p]).wait()

        def start_write(f, p):
            for tr in range(D // 8):
                off = f * FPLANE + tr * (B // 128) * 1024 + tc0 * 1024
                pltpu.async_copy(
                    T[p].at[pl.ds(tr * TBLK, TBLK)],
                    out_hbm.at[pl.ds(off, TBLK)],
                    wsem[p],
                )

        def wait_write(p):
            for tr in range(D // 8):
                pltpu.make_async_copy(
                    T[p].at[pl.ds(tr * TBLK, TBLK)],
                    out_hbm.at[pl.ds(0, TBLK)],
                    wsem[p],
                ).wait()

        def transpose(p):
            g, t = G[p], T[p]

            @plsc.parallel_loop(0, 512, unroll=4)
            def body(i):
                s = i & 15
                r0 = i >> 4 << 4
                dp = (iota16 + s) & 15
                tb = (dp >> 3) * TBLK + (dp & 7) * 128 + iota16
                rows = r0 + iota16
                off = (r0 >> 7) * 1024 + (r0 & 127)
                lo = plsc.load_gather(g, [rows, dp])
                hi = plsc.load_gather(g, [rows, dp + 16])
                plsc.store_scatter(t, [tb + off], lo)
                plsc.store_scatter(t, [tb + (off + 2 * TBLK)], hi)

        def step(f, p, *, do_next_gather, do_wait_write):
            if do_next_gather:
                extract(f + 1, 1 - p)
                start_gather(1 - p)
            wait_gather(p)
            if do_wait_write:
                wait_write(p)
            transpose(p)
            start_write(f, p)

        pltpu.sync_copy(x_hbm.at[pl.ds(b0, BPW), :], xblk)
        extract(0, 0)
        start_gather(0)

        step(0, 0, do_next_gather=True, do_wait_write=False)
        step(1, 1, do_next_gather=True, do_wait_write=False)

        def body(h, carry):
            f = h * 2
            step(f, 0, do_next_gather=True, do_wait_write=True)
            step(f + 1, 1, do_next_gather=True, do_wait_write=True)
            return carry

        lax.fori_loop(1, F // 2 - 1, body, 0)

        step(F - 2, 0, do_next_gather=True, do_wait_write=True)
        step(F - 1, 1, do_next_gather=False, do_wait_write=True)

        wait_write(0)
        wait_write(1)

    return k


def kernel(x, table):
    info = plsc.get_sparse_core_info()
    out1 = _build_call(info.num_cores, info.num_subcores)(
        x.astype(jnp.int32), table
    )
    out5 = jnp.reshape(out1, (F, D // 8, B // 128, 8, 128))
    return jnp.reshape(jnp.transpose(out5, (2, 4, 0, 1, 3)), (B, F, D))


# speedup vs baseline: 1.8385x; 1.8385x over previous
import functools

import jax
import jax.numpy as jnp
from jax import lax
from jax.experimental import pallas as pl
from jax.experimental.pallas import tpu as pltpu
from jax.experimental.pallas import tpu_sc as plsc

B = 16384
F = 100
D = 32
BPW = 512
TCW = BPW // 128
TBLK = TCW * 8 * 128
FPLANE = (D // 8) * (B // 128) * 8 * 128


NROW = 1000000
NTILE = NROW // 128
KFULL = 2 * NTILE // 64


def _build_detile(num_cores, num_subcores):
    mesh = plsc.VectorSubcoreMesh(core_axis_name="c", subcore_axis_name="s")

    @functools.partial(
        pl.kernel,
        mesh=mesh,
        out_type=jax.ShapeDtypeStruct((NROW * D,), jnp.float32),
        scratch_types=[
            pltpu.VMEM((D, 128), jnp.float32),
            pltpu.VMEM((D, 128), jnp.float32),
            pltpu.VMEM((128 * D,), jnp.float32),
            pltpu.VMEM((128 * D,), jnp.float32),
            pltpu.VMEM((D, 64), jnp.float32),
            pltpu.VMEM((64 * D,), jnp.float32),
            pltpu.SemaphoreType.DMA,
            pltpu.SemaphoreType.DMA,
            pltpu.SemaphoreType.DMA,
            pltpu.SemaphoreType.DMA,
        ],
        compiler_params=pltpu.CompilerParams(
            use_tc_tiling_on_sc=True, needs_layout_passes=False
        ),
    )
    def k(tabt_hbm, out_hbm, s0, s1, d0, d1, s64, d64, is0, is1, os0, os1):
        wid = lax.axis_index("s") * num_cores + lax.axis_index("c")
        S = (s0, s1)
        DD = (d0, d1)
        isem = (is0, is1)
        osem = (os0, os1)
        nw = num_cores * num_subcores

        iota16 = lax.iota(jnp.int32, 16)

        def chunk(kk):
            return wid + kk * nw

        def start_in(kk, p):
            c0 = pl.multiple_of(chunk(kk) * 128, 128)
            pltpu.async_copy(tabt_hbm.at[:, pl.ds(c0, 128)], S[p], isem[p])

        def wait_in(p):
            pltpu.make_async_copy(
                tabt_hbm.at[:, pl.ds(0, 128)], S[p], isem[p]
            ).wait()

        def start_out(kk, p):
            off = pl.multiple_of(chunk(kk) * (128 * D), 128 * D)
            pltpu.async_copy(DD[p], out_hbm.at[pl.ds(off, 128 * D)], osem[p])

        def wait_out(p):
            pltpu.make_async_copy(
                DD[p], out_hbm.at[pl.ds(0, 128 * D)], osem[p]
            ).wait()

        def transpose(src, dst, ncol):
            nblk = (D // 16) * (ncol // 16) * 16

            @plsc.parallel_loop(0, nblk, unroll=4)
            def body(i):
                s = i & 15
                cb = (i >> 4) & (ncol // 16 - 1)
                db = i >> 4 >> (ncol // 16).bit_length() - 1
                dvec = ((iota16 + s) & 15) + db * 16
                colv = cb * 16 + iota16
                v = plsc.load_gather(src, [dvec, colv])
                plsc.store_scatter(dst, [colv * D + dvec], v)

        def step(kk, p, *, prefetch):
            wait_in(p)
            wait_out(p)
            transpose(S[p], DD[p], 128)
            start_out(kk, p)
            if prefetch:
                start_in(kk + 2, p)

        start_in(0, 0)
        start_in(1, 1)
        wait_in(0)
        transpose(S[0], DD[0], 128)
        start_out(0, 0)
        start_in(2, 0)
        wait_in(1)
        transpose(S[1], DD[1], 128)
        start_out(1, 1)
        start_in(3, 1)

        def body(h, carry):
            step(2 * h, 0, prefetch=True)
            step(2 * h + 1, 1, prefetch=True)
            return carry

        lax.fori_loop(1, KFULL // 2 - 1, body, 0)

        step(KFULL - 2, 0, prefetch=False)
        step(KFULL - 1, 1, prefetch=False)
        wait_out(0)
        wait_out(1)

        @pl.when(wid < (NTILE - KFULL * nw))
        def _():
            c0 = pl.multiple_of((KFULL * nw + wid) * 128, 128)
            pltpu.sync_copy(tabt_hbm.at[:, pl.ds(c0, 128)], S[0])
            transpose(S[0], DD[0], 128)
            pltpu.sync_copy(DD[0], out_hbm.at[pl.ds(c0 * D, 128 * D)])

        @pl.when(wid == (NTILE - KFULL * nw))
        def _():
            pltpu.sync_copy(tabt_hbm.at[:, pl.ds(NTILE * 128, 64)], s64)
            transpose(s64, d64, 64)
            pltpu.sync_copy(d64, out_hbm.at[pl.ds(NTILE * 128 * D, 64 * D)])

    return k


def _build_call(num_cores, num_subcores):
    mesh = plsc.VectorSubcoreMesh(core_axis_name="c", subcore_axis_name="s")

    @functools.partial(
        pl.kernel,
        mesh=mesh,
        out_type=jax.ShapeDtypeStruct((F * FPLANE,), jnp.float32),
        scratch_types=[
            pltpu.VMEM((BPW, F), jnp.int32),
            pltpu.VMEM((BPW,), jnp.int32),
            pltpu.VMEM((BPW,), jnp.int32),
            pltpu.VMEM((BPW, D), jnp.float32),
            pltpu.VMEM((BPW, D), jnp.float32),
            pltpu.VMEM(((D // 8) * TBLK,), jnp.float32),
            pltpu.VMEM(((D // 8) * TBLK,), jnp.float32),
            pltpu.SemaphoreType.DMA,
            pltpu.SemaphoreType.DMA,
            pltpu.SemaphoreType.DMA,
            pltpu.SemaphoreType.DMA,
        ],
        compiler_params=pltpu.CompilerParams(
            use_tc_tiling_on_sc=False, needs_layout_passes=False
        ),
    )
    def k(x_hbm, tab_hbm, out_hbm, xblk, if0, if1, g0, g1, t0, t1, gs0, gs1, ws0, ws1):
        wid = lax.axis_index("s") * num_cores + lax.axis_index("c")
        b0 = pl.multiple_of(wid * BPW, BPW)
        tc0 = wid * TCW
        IF = (if0, if1)
        G = (g0, g1)
        T = (t0, t1)
        gsem = (gs0, gs1)
        wsem = (ws0, ws1)

        iota16 = lax.iota(jnp.int32, 16)

        def extract(f, p):
            dst = IF[p]
            fvec = iota16 * 0 + f

            for j in range(BPW // 16):
                rows = j * 16 + iota16
                v = plsc.load_gather(xblk, [rows, fvec])
                plsc.store_scatter(dst, [rows], v)

        def start_gather(p):
            pltpu.async_copy(tab_hbm.at[IF[p]], G[p], gsem[p])

        def wait_gather(p):
            pltpu.make_async_copy(tab_hbm.at[IF[p]], G[p], gsem[p]).wait()

        def start_write(f, p):
            for tr in range(D // 8):
                off = f * FPLANE + tr * (B // 128) * 1024 + tc0 * 1024
                pltpu.async_copy(
                    T[p].at[pl.ds(tr * TBLK, TBLK)],
                    out_hbm.at[pl.ds(off, TBLK)],
                    wsem[p],
                )

        def wait_write(p):
            for tr in range(D // 8):
                pltpu.make_async_copy(
                    T[p].at[pl.ds(tr * TBLK, TBLK)],
                    out_hbm.at[pl.ds(0, TBLK)],
                    wsem[p],
                ).wait()

        def transpose(p):
            g, t = G[p], T[p]

            @plsc.parallel_loop(0, 512, unroll=4)
            def body(i):
                s = i & 15
                r0 = i >> 4 << 4
                dp = (iota16 + s) & 15
                tb = (dp >> 3) * TBLK + (dp & 7) * 128 + iota16
                rows = r0 + iota16
                off = (r0 >> 7) * 1024 + (r0 & 127)
                lo = plsc.load_gather(g, [rows, dp])
                hi = plsc.load_gather(g, [rows, dp + 16])
                plsc.store_scatter(t, [tb + off], lo)
                plsc.store_scatter(t, [tb + (off + 2 * TBLK)], hi)

        def step(f, p, *, do_next_gather, do_wait_write):
            if do_next_gather:
                extract(f + 1, 1 - p)
                start_gather(1 - p)
            wait_gather(p)
            if do_wait_write:
                wait_write(p)
            transpose(p)
            start_write(f, p)

        pltpu.sync_copy(x_hbm.at[pl.ds(b0, BPW), :], xblk)
        extract(0, 0)
        start_gather(0)

        step(0, 0, do_next_gather=True, do_wait_write=False)
        step(1, 1, do_next_gather=True, do_wait_write=False)

        def body(h, carry):
            f = h * 2
            step(f, 0, do_next_gather=True, do_wait_write=True)
            step(f + 1, 1, do_next_gather=True, do_wait_write=True)
            return carry

        lax.fori_loop(1, F // 2 - 1, body, 0)

        step(F - 2, 0, do_next_gather=True, do_wait_write=True)
        step(F - 1, 1, do_next_gather=False, do_wait_write=True)

        wait_write(0)
        wait_write(1)

    return k


def kernel(x, table):
    info = plsc.get_sparse_core_info()
    tab_lin = _build_detile(info.num_cores, info.num_subcores)(
        jnp.transpose(table)
    )
    out1 = _build_call(info.num_cores, info.num_subcores)(
        x.astype(jnp.int32), jnp.reshape(tab_lin, (NROW, D))
    )
    out5 = jnp.reshape(out1, (F, D // 8, B // 128, 8, 128))
    return jnp.reshape(jnp.transpose(out5, (2, 4, 0, 1, 3)), (B, F, D))
